# Initial kernel scaffold; baseline (speedup 1.0000x reference)
import jax
import jax.numpy as jnp
from jax.experimental import pallas as pl


def kernel(prop_atom_in_ring, prop_atom_charge, prop_atom_hybridization, prop_atom_chirality, W_in_ring, W_charge, W_hybridization, W_chirality):
    raise NotImplementedError("write your pallas kernel here")


# speedup vs baseline: 14.6694x; 14.6694x over previous
import functools

import jax
import jax.numpy as jnp
from jax import lax
from jax.experimental import pallas as pl
from jax.experimental.pallas import tpu as pltpu
from jax.experimental.pallas import tpu_sc as plsc

_B, _L, _D = 1024, 200, 128
_BL = _B * _L
_N_RING, _N_CHARGE, _N_HYB, _N_CHIR = 3, 4, 9, 5
_NCOMB = _N_RING * _N_CHARGE * _N_HYB * _N_CHIR
_NCOMB_PAD = 544

_NC, _NS = 2, 16
_NW = _NC * _NS
_PER_W = _BL // _NW
_CHUNK = 128
_NCHUNKS = _PER_W // _CHUNK


def _prologue_body(r_ref, c_ref, h_ref, x_ref,
                   wr_ref, wc_ref, wh_ref, wx_ref,
                   cidx_ref, wcomb_ref):
    cidx_ref[...] = (
        (r_ref[...] * (_N_CHARGE * _N_HYB * _N_CHIR)
         + c_ref[...] * (_N_HYB * _N_CHIR)
         + h_ref[...] * _N_CHIR)
        + x_ref[...]
    )
    i = lax.broadcasted_iota(jnp.int32, (_NCOMB_PAD, 1), 0)
    r = i // (_N_CHARGE * _N_HYB * _N_CHIR)
    c = (i // (_N_HYB * _N_CHIR)) % _N_CHARGE
    h = (i // _N_CHIR) % _N_HYB
    x = i % _N_CHIR

    def pick(idx_col, tab_ref):
        n = tab_ref.shape[0]
        cols = lax.broadcasted_iota(jnp.int32, (_NCOMB_PAD, n), 1)
        onehot = (idx_col == cols).astype(jnp.float32)
        return jnp.dot(onehot, tab_ref[...], preferred_element_type=jnp.float32)

    wcomb_ref[...] = (pick(r, wr_ref) + pick(c, wc_ref)
                      + pick(h, wh_ref) + pick(x, wx_ref))


def _sc_gather_body(cidx_hbm, wcomb_hbm, out_hbm, idx_v, rows_v, sem):
    wid = lax.axis_index("s") * _NC + lax.axis_index("c")
    tile_base = wid * _PER_W

    def chunk(g, carry):
        base = tile_base + g * _CHUNK
        pltpu.sync_copy(cidx_hbm.at[pl.ds(base, _CHUNK)], idx_v)
        pltpu.async_copy(wcomb_hbm.at[idx_v], rows_v, sem).wait()
        pltpu.sync_copy(rows_v, out_hbm.at[pl.ds(base, _CHUNK)])
        return carry

    lax.fori_loop(0, _NCHUNKS, chunk, 0)


_sc_gather = functools.partial(
    pl.kernel,
    out_type=jax.ShapeDtypeStruct((_BL, _D), jnp.float32),
    mesh=plsc.VectorSubcoreMesh(core_axis_name="c", subcore_axis_name="s"),
    scratch_types=[
        pltpu.VMEM((_CHUNK,), jnp.int32),
        pltpu.VMEM((_CHUNK, _D), jnp.float32),
        pltpu.SemaphoreType.DMA,
    ],
)(_sc_gather_body)


@jax.jit
def kernel(prop_atom_in_ring, prop_atom_charge, prop_atom_hybridization,
           prop_atom_chirality, W_in_ring, W_charge, W_hybridization,
           W_chirality):
    r = prop_atom_in_ring.astype(jnp.int32)
    c = prop_atom_charge.astype(jnp.int32)
    h = prop_atom_hybridization.astype(jnp.int32)
    x = prop_atom_chirality.astype(jnp.int32)
    wr = jnp.pad(W_in_ring, ((0, 8 - _N_RING), (0, 0)))
    wc = jnp.pad(W_charge, ((0, 8 - _N_CHARGE), (0, 0)))
    wh = jnp.pad(W_hybridization, ((0, 16 - _N_HYB), (0, 0)))
    wx = jnp.pad(W_chirality, ((0, 8 - _N_CHIR), (0, 0)))

    cidx, wcomb = pl.pallas_call(
        _prologue_body,
        out_shape=(
            jax.ShapeDtypeStruct((_B, _L), jnp.int32),
            jax.ShapeDtypeStruct((_NCOMB_PAD, _D), jnp.float32),
        ),
    )(r, c, h, x, wr, wc, wh, wx)

    out = _sc_gather(cidx.reshape(_BL), wcomb)
    return out.reshape(_B, _L, _D)


# speedup vs baseline: 15.8948x; 1.0835x over previous
import functools

import jax
import jax.numpy as jnp
from jax import lax
from jax.experimental import pallas as pl
from jax.experimental.pallas import tpu as pltpu
from jax.experimental.pallas import tpu_sc as plsc

_B, _L, _D = 1024, 200, 128
_BL = _B * _L
_N_RING, _N_CHARGE, _N_HYB, _N_CHIR = 3, 4, 9, 5
_NCOMB = _N_RING * _N_CHARGE * _N_HYB * _N_CHIR
_NCOMB_PAD = 544

_NC, _NS = 2, 16
_NW = _NC * _NS
_PER_W = _BL // _NW
_CHUNK = 128
_NCHUNKS = _PER_W // _CHUNK


def _prologue_body(r_ref, c_ref, h_ref, x_ref,
                   wr_ref, wc_ref, wh_ref, wx_ref,
                   cidx_ref, wcomb_ref):
    cidx_ref[...] = (
        (r_ref[...] * (_N_CHARGE * _N_HYB * _N_CHIR)
         + c_ref[...] * (_N_HYB * _N_CHIR)
         + h_ref[...] * _N_CHIR)
        + x_ref[...]
    )
    i = lax.broadcasted_iota(jnp.int32, (_NCOMB_PAD, 1), 0)
    r = i // (_N_CHARGE * _N_HYB * _N_CHIR)
    c = (i // (_N_HYB * _N_CHIR)) % _N_CHARGE
    h = (i // _N_CHIR) % _N_HYB
    x = i % _N_CHIR

    def pick(idx_col, tab_ref):
        n = tab_ref.shape[0]
        cols = lax.broadcasted_iota(jnp.int32, (_NCOMB_PAD, n), 1)
        onehot = (idx_col == cols).astype(jnp.float32)
        return jnp.dot(onehot, tab_ref[...], preferred_element_type=jnp.float32)

    wcomb_ref[...] = (pick(r, wr_ref) + pick(c, wc_ref)
                      + pick(h, wh_ref) + pick(x, wx_ref))


_NBUF = 2


def _sc_gather_body(cidx_hbm, wcomb_hbm, out_hbm, idx_v, rows_v, gsem, s0, s1):
    wid = lax.axis_index("s") * _NC + lax.axis_index("c")
    tile_base = wid * _PER_W
    pltpu.sync_copy(cidx_hbm.at[pl.ds(tile_base, _PER_W)], idx_v)
    ssems = [s0, s1]

    def scat(b, g):
        base = tile_base + g * _CHUNK
        return pltpu.make_async_copy(
            rows_v.at[b], out_hbm.at[pl.ds(base, _CHUNK)], ssems[b])

    def step(i, carry):
        for b in range(_NBUF):
            g = i * _NBUF + b

            @pl.when(i > 0)
            def _():
                scat(b, g).wait()
            pltpu.async_copy(
                wcomb_hbm.at[idx_v.at[pl.ds(g * _CHUNK, _CHUNK)]],
                rows_v.at[b], gsem).wait()
            scat(b, g).start()
        return carry

    lax.fori_loop(0, _NCHUNKS // _NBUF, step, 0)
    for b in range(_NBUF):
        scat(b, 0).wait()


_sc_gather = functools.partial(
    pl.kernel,
    out_type=jax.ShapeDtypeStruct((_BL, _D), jnp.float32),
    mesh=plsc.VectorSubcoreMesh(core_axis_name="c", subcore_axis_name="s"),
    scratch_types=[
        pltpu.VMEM((_PER_W,), jnp.int32),
        pltpu.VMEM((_NBUF, _CHUNK, _D), jnp.float32),
        pltpu.SemaphoreType.DMA,
        pltpu.SemaphoreType.DMA,
        pltpu.SemaphoreType.DMA,
    ],
)(_sc_gather_body)


@jax.jit
def kernel(prop_atom_in_ring, prop_atom_charge, prop_atom_hybridization,
           prop_atom_chirality, W_in_ring, W_charge, W_hybridization,
           W_chirality):
    r = prop_atom_in_ring.astype(jnp.int32)
    c = prop_atom_charge.astype(jnp.int32)
    h = prop_atom_hybridization.astype(jnp.int32)
    x = prop_atom_chirality.astype(jnp.int32)
    wr = jnp.pad(W_in_ring, ((0, 8 - _N_RING), (0, 0)))
    wc = jnp.pad(W_charge, ((0, 8 - _N_CHARGE), (0, 0)))
    wh = jnp.pad(W_hybridization, ((0, 16 - _N_HYB), (0, 0)))
    wx = jnp.pad(W_chirality, ((0, 8 - _N_CHIR), (0, 0)))

    cidx, wcomb = pl.pallas_call(
        _prologue_body,
        out_shape=(
            jax.ShapeDtypeStruct((_B, _L), jnp.int32),
            jax.ShapeDtypeStruct((_NCOMB_PAD, _D), jnp.float32),
        ),
    )(r, c, h, x, wr, wc, wh, wx)

    out = _sc_gather(cidx.reshape(_BL), wcomb)
    return out.reshape(_B, _L, _D)


# speedup vs baseline: 16.1746x; 1.0176x over previous
import functools

import jax
import jax.numpy as jnp
from jax import lax
from jax.experimental import pallas as pl
from jax.experimental.pallas import tpu as pltpu
from jax.experimental.pallas import tpu_sc as plsc

_B, _L, _D = 1024, 200, 128
_BL = _B * _L
_N_RING, _N_CHARGE, _N_HYB, _N_CHIR = 3, 4, 9, 5
_NCOMB = _N_RING * _N_CHARGE * _N_HYB * _N_CHIR
_NCOMB_PAD = 544

_NC, _NS = 2, 16
_NW = _NC * _NS
_PER_W = _BL // _NW
_CHUNK = 128
_NCHUNKS = _PER_W // _CHUNK


def _prologue_body(r_ref, c_ref, h_ref, x_ref,
                   wr_ref, wc_ref, wh_ref, wx_ref,
                   cidx_ref, wcomb_ref):
    cidx_ref[...] = (
        (r_ref[...] * (_N_CHARGE * _N_HYB * _N_CHIR)
         + c_ref[...] * (_N_HYB * _N_CHIR)
         + h_ref[...] * _N_CHIR)
        + x_ref[...]
    )
    i = lax.broadcasted_iota(jnp.int32, (_NCOMB_PAD, 1), 0)
    r = i // (_N_CHARGE * _N_HYB * _N_CHIR)
    c = (i // (_N_HYB * _N_CHIR)) % _N_CHARGE
    h = (i // _N_CHIR) % _N_HYB
    x = i % _N_CHIR

    def pick(idx_col, tab_ref):
        n = tab_ref.shape[0]
        cols = lax.broadcasted_iota(jnp.int32, (_NCOMB_PAD, n), 1)
        onehot = (idx_col == cols).astype(jnp.float32)
        return jnp.dot(onehot, tab_ref[...], preferred_element_type=jnp.float32)

    wcomb_ref[...] = (pick(r, wr_ref) + pick(c, wc_ref)
                      + pick(h, wh_ref) + pick(x, wx_ref))


_NBUF = 2


def _sc_gather_body(cidx_hbm, wcomb_hbm, out_hbm, idx_v, rows_v, g0, g1, s0, s1):
    wid = lax.axis_index("s") * _NC + lax.axis_index("c")
    tile_base = wid * _PER_W
    pltpu.sync_copy(cidx_hbm.at[pl.ds(tile_base, _PER_W)], idx_v)
    ssems = [s0, s1]
    gsems = [g0, g1]

    def scat(b, g):
        base = tile_base + g * _CHUNK
        return pltpu.make_async_copy(
            rows_v.at[b], out_hbm.at[pl.ds(base, _CHUNK)], ssems[b])

    def gath(b, g):
        return pltpu.make_async_copy(
            wcomb_hbm.at[idx_v.at[pl.ds(g * _CHUNK, _CHUNK)]],
            rows_v.at[b], gsems[b])

    def step(i, carry):
        for b in range(_NBUF):
            g = i * _NBUF + b

            @pl.when(i > 0)
            def _():
                scat(b, g - 2).wait()
                if b == 0:
                    gath(b, g).start()
                    gath(1, g - 1).wait()
                    scat(1, g - 1).start()

            if b == 0:
                @pl.when(i == 0)
                def _():
                    gath(b, g).start()
            else:
                gath(b, g).start()
                gath(0, g - 1).wait()
                scat(0, g - 1).start()
        return carry

    lax.fori_loop(0, _NCHUNKS // _NBUF, step, 0)
    gath(1, _NCHUNKS - 1).wait()
    scat(1, _NCHUNKS - 1).start()
    scat(0, 0).wait()
    scat(1, 0).wait()


_sc_gather = functools.partial(
    pl.kernel,
    out_type=jax.ShapeDtypeStruct((_BL, _D), jnp.float32),
    mesh=plsc.VectorSubcoreMesh(core_axis_name="c", subcore_axis_name="s"),
    scratch_types=[
        pltpu.VMEM((_PER_W,), jnp.int32),
        pltpu.VMEM((_NBUF, _CHUNK, _D), jnp.float32),
        pltpu.SemaphoreType.DMA,
        pltpu.SemaphoreType.DMA,
        pltpu.SemaphoreType.DMA,
        pltpu.SemaphoreType.DMA,
    ],
)(_sc_gather_body)


@jax.jit
def kernel(prop_atom_in_ring, prop_atom_charge, prop_atom_hybridization,
           prop_atom_chirality, W_in_ring, W_charge, W_hybridization,
           W_chirality):
    r = prop_atom_in_ring.astype(jnp.int32)
    c = prop_atom_charge.astype(jnp.int32)
    h = prop_atom_hybridization.astype(jnp.int32)
    x = prop_atom_chirality.astype(jnp.int32)
    wr = jnp.pad(W_in_ring, ((0, 8 - _N_RING), (0, 0)))
    wc = jnp.pad(W_charge, ((0, 8 - _N_CHARGE), (0, 0)))
    wh = jnp.pad(W_hybridization, ((0, 16 - _N_HYB), (0, 0)))
    wx = jnp.pad(W_chirality, ((0, 8 - _N_CHIR), (0, 0)))

    cidx, wcomb = pl.pallas_call(
        _prologue_body,
        out_shape=(
            jax.ShapeDtypeStruct((_B, _L), jnp.int32),
            jax.ShapeDtypeStruct((_NCOMB_PAD, _D), jnp.float32),
        ),
    )(r, c, h, x, wr, wc, wh, wx)

    out = _sc_gather(cidx.reshape(_BL), wcomb)
    return out.reshape(_B, _L, _D)


# speedup vs baseline: 34.9611x; 2.1615x over previous
import functools

import jax
import jax.numpy as jnp
from jax import lax
from jax.experimental import pallas as pl
from jax.experimental.pallas import tpu as pltpu
from jax.experimental.pallas import tpu_sc as plsc

_B, _L, _D = 1024, 200, 128
_BL = _B * _L
_N_RING, _N_CHARGE, _N_HYB, _N_CHIR = 3, 4, 9, 5
_NCOMB = _N_RING * _N_CHARGE * _N_HYB * _N_CHIR
_NCOMB_PAD = 544

_NC, _NS = 2, 16
_NW = _NC * _NS
_PER_W = _BL // _NW
_CHUNK = 128
_NCHUNKS = _PER_W // _CHUNK


def _prologue_body(r_ref, c_ref, h_ref, x_ref,
                   wr_ref, wc_ref, wh_ref, wx_ref,
                   cidx_ref, wcomb_ref):
    cidx_ref[...] = (
        (r_ref[...] * (_N_CHARGE * _N_HYB * _N_CHIR)
         + c_ref[...] * (_N_HYB * _N_CHIR)
         + h_ref[...] * _N_CHIR)
        + x_ref[...]
    )
    i = lax.broadcasted_iota(jnp.int32, (_NCOMB_PAD, 1), 0)
    r = i // (_N_CHARGE * _N_HYB * _N_CHIR)
    c = (i // (_N_HYB * _N_CHIR)) % _N_CHARGE
    h = (i // _N_CHIR) % _N_HYB
    x = i % _N_CHIR

    def pick(idx_col, tab_ref):
        n = tab_ref.shape[0]
        cols = lax.broadcasted_iota(jnp.int32, (_NCOMB_PAD, n), 1)
        onehot = (idx_col == cols).astype(jnp.float32)
        return jnp.dot(onehot, tab_ref[...], preferred_element_type=jnp.float32)

    wcomb_ref[...] = (pick(r, wr_ref) + pick(c, wc_ref)
                      + pick(h, wh_ref) + pick(x, wx_ref))


_NBUF = 2


def _sc_gather_body(cidx_hbm, wcomb_hbm, out_hbm, idx_v, rows_v, wcomb_sh,
                    g0, g1, s0, s1):
    wid = lax.axis_index("s") * _NC + lax.axis_index("c")
    tile_base = wid * _PER_W
    @pl.when(lax.axis_index("s") == 0)
    def _():
        pltpu.sync_copy(wcomb_hbm, wcomb_sh)

    pltpu.sync_copy(cidx_hbm.at[pl.ds(tile_base, _PER_W)], idx_v)
    plsc.subcore_barrier()
    ssems = [s0, s1]
    gsems = [g0, g1]

    def scat(b, g):
        base = tile_base + g * _CHUNK
        return pltpu.make_async_copy(
            rows_v.at[b], out_hbm.at[pl.ds(base, _CHUNK)], ssems[b])

    def gath(b, g):
        return pltpu.make_async_copy(
            wcomb_sh.at[idx_v.at[pl.ds(g * _CHUNK, _CHUNK)]],
            rows_v.at[b], gsems[b])

    def step(i, carry):
        for b in range(_NBUF):
            g = i * _NBUF + b

            @pl.when(i > 0)
            def _():
                scat(b, g - 2).wait()
                if b == 0:
                    gath(b, g).start()
                    gath(1, g - 1).wait()
                    scat(1, g - 1).start()

            if b == 0:
                @pl.when(i == 0)
                def _():
                    gath(b, g).start()
            else:
                gath(b, g).start()
                gath(0, g - 1).wait()
                scat(0, g - 1).start()
        return carry

    lax.fori_loop(0, _NCHUNKS // _NBUF, step, 0)
    gath(1, _NCHUNKS - 1).wait()
    scat(1, _NCHUNKS - 1).start()
    scat(0, 0).wait()
    scat(1, 0).wait()


_sc_gather = functools.partial(
    pl.kernel,
    out_type=jax.ShapeDtypeStruct((_BL, _D), jnp.float32),
    mesh=plsc.VectorSubcoreMesh(core_axis_name="c", subcore_axis_name="s"),
    scratch_types=[
        pltpu.VMEM((_PER_W,), jnp.int32),
        pltpu.VMEM((_NBUF, _CHUNK, _D), jnp.float32),
        pltpu.VMEM_SHARED((_NCOMB_PAD, _D), jnp.float32),
        pltpu.SemaphoreType.DMA,
        pltpu.SemaphoreType.DMA,
        pltpu.SemaphoreType.DMA,
        pltpu.SemaphoreType.DMA,
    ],
)(_sc_gather_body)


@jax.jit
def kernel(prop_atom_in_ring, prop_atom_charge, prop_atom_hybridization,
           prop_atom_chirality, W_in_ring, W_charge, W_hybridization,
           W_chirality):
    r = prop_atom_in_ring.astype(jnp.int32)
    c = prop_atom_charge.astype(jnp.int32)
    h = prop_atom_hybridization.astype(jnp.int32)
    x = prop_atom_chirality.astype(jnp.int32)
    wr = jnp.pad(W_in_ring, ((0, 8 - _N_RING), (0, 0)))
    wc = jnp.pad(W_charge, ((0, 8 - _N_CHARGE), (0, 0)))
    wh = jnp.pad(W_hybridization, ((0, 16 - _N_HYB), (0, 0)))
    wx = jnp.pad(W_chirality, ((0, 8 - _N_CHIR), (0, 0)))

    cidx, wcomb = pl.pallas_call(
        _prologue_body,
        out_shape=(
            jax.ShapeDtypeStruct((_B, _L), jnp.int32),
            jax.ShapeDtypeStruct((_NCOMB_PAD, _D), jnp.float32),
        ),
    )(r, c, h, x, wr, wc, wh, wx)

    out = _sc_gather(cidx.reshape(_BL), wcomb)
    return out.reshape(_B, _L, _D)


# speedup vs baseline: 34.9729x; 1.0003x over previous
import functools

import jax
import jax.numpy as jnp
from jax import lax
from jax.experimental import pallas as pl
from jax.experimental.pallas import tpu as pltpu
from jax.experimental.pallas import tpu_sc as plsc

_B, _L, _D = 1024, 200, 128
_BL = _B * _L
_N_RING, _N_CHARGE, _N_HYB, _N_CHIR = 3, 4, 9, 5
_NCOMB_PAD = 544

_NC, _NS = 2, 16
_NW = _NC * _NS
_PER_W = _BL // _NW
_CHUNK = 128
_NCHUNKS = _PER_W // _CHUNK
_ROWS_PER_TILE = _NCOMB_PAD // _NS
_NBUF = 2


def _sc_body(ring_hbm, charge_hbm, hyb_hbm, chir_hbm,
             wr_hbm, wc_hbm, wh_hbm, wx_hbm,
             out_hbm,
             idx4_v, cidx_v, wr_v, wc_v, wh_v, wx_v, tmp_v, rows_v, wcomb_sh,
             isem, g0, g1, s0, s1):
    cid = lax.axis_index("c")
    sid = lax.axis_index("s")
    wid = sid * _NC + cid
    tile_base = wid * _PER_W

    idx_cp = [
        pltpu.make_async_copy(src.at[pl.ds(tile_base, _PER_W)],
                              idx4_v.at[i], isem)
        for i, src in enumerate((ring_hbm, charge_hbm, hyb_hbm, chir_hbm))
    ]
    for cp in idx_cp:
        cp.start()

    tabs = [wr_v, wc_v, wh_v, wx_v]
    for dst, s_ in zip(tabs, (wr_hbm, wc_hbm, wh_hbm, wx_hbm)):
        pltpu.sync_copy(s_, dst.at[pl.ds(0, s_.shape[0])])

    def build_row(jl, carry):
        j = sid * _ROWS_PER_TILE + jl
        r = j // (_N_CHARGE * _N_HYB * _N_CHIR)
        c = (j // (_N_HYB * _N_CHIR)) % _N_CHARGE
        h = (j // _N_CHIR) % _N_HYB
        x = j % _N_CHIR
        for k in range(_D // 16):
            v = (wr_v[pl.ds(r * _D + 16 * k, 16)]
                 + wc_v[pl.ds(c * _D + 16 * k, 16)]
                 + wh_v[pl.ds(h * _D + 16 * k, 16)]
                 + wx_v[pl.ds(x * _D + 16 * k, 16)])
            tmp_v[jl, pl.ds(16 * k, 16)] = v
        return carry

    lax.fori_loop(0, _ROWS_PER_TILE, build_row, 0)
    pltpu.sync_copy(
        tmp_v, wcomb_sh.at[pl.ds(sid * _ROWS_PER_TILE, _ROWS_PER_TILE)])

    for cp in idx_cp:
        cp.wait()

    def fuse(i, carry):
        s = pl.ds(i * 16, 16)
        cidx_v[s] = ((idx4_v[0, s] * (_N_CHARGE * _N_HYB * _N_CHIR))
                     + (idx4_v[1, s] * (_N_HYB * _N_CHIR))
                     + (idx4_v[2, s] * _N_CHIR)
                     + idx4_v[3, s])
        return carry

    lax.fori_loop(0, _PER_W // 16, fuse, 0)

    plsc.subcore_barrier()

    ssems = [s0, s1]
    gsems = [g0, g1]

    def scat(b, g):
        base = tile_base + g * _CHUNK
        return pltpu.make_async_copy(
            rows_v.at[b], out_hbm.at[pl.ds(base, _CHUNK)], ssems[b])

    def gath(b, g):
        return pltpu.make_async_copy(
            wcomb_sh.at[cidx_v.at[pl.ds(g * _CHUNK, _CHUNK)]],
            rows_v.at[b], gsems[b])

    def step(i, carry):
        for b in range(_NBUF):
            g = i * _NBUF + b

            @pl.when(i > 0)
            def _():
                scat(b, g - 2).wait()
                if b == 0:
                    gath(b, g).start()
                    gath(1, g - 1).wait()
                    scat(1, g - 1).start()

            if b == 0:
                @pl.when(i == 0)
                def _():
                    gath(b, g).start()
            else:
                gath(b, g).start()
                gath(0, g - 1).wait()
                scat(0, g - 1).start()
        return carry

    lax.fori_loop(0, _NCHUNKS // _NBUF, step, 0)
    gath(1, _NCHUNKS - 1).wait()
    scat(1, _NCHUNKS - 1).start()
    scat(0, 0).wait()
    scat(1, 0).wait()


_sc_kernel = functools.partial(
    pl.kernel,
    out_type=jax.ShapeDtypeStruct((_BL, _D), jnp.float32),
    mesh=plsc.VectorSubcoreMesh(core_axis_name="c", subcore_axis_name="s"),
    scratch_types=[
        pltpu.VMEM((4, _PER_W), jnp.int32),
        pltpu.VMEM((_PER_W,), jnp.int32),
        pltpu.VMEM((16 * _D,), jnp.float32),
        pltpu.VMEM((16 * _D,), jnp.float32),
        pltpu.VMEM((16 * _D,), jnp.float32),
        pltpu.VMEM((16 * _D,), jnp.float32),
        pltpu.VMEM((_ROWS_PER_TILE, _D), jnp.float32),
        pltpu.VMEM((_NBUF, _CHUNK, _D), jnp.float32),
        pltpu.VMEM_SHARED((_NCOMB_PAD, _D), jnp.float32),
        pltpu.SemaphoreType.DMA,
        pltpu.SemaphoreType.DMA,
        pltpu.SemaphoreType.DMA,
        pltpu.SemaphoreType.DMA,
        pltpu.SemaphoreType.DMA,
    ],
)(_sc_body)


@jax.jit
def kernel(prop_atom_in_ring, prop_atom_charge, prop_atom_hybridization,
           prop_atom_chirality, W_in_ring, W_charge, W_hybridization,
           W_chirality):
    r = prop_atom_in_ring.astype(jnp.int32).reshape(_BL)
    c = prop_atom_charge.astype(jnp.int32).reshape(_BL)
    h = prop_atom_hybridization.astype(jnp.int32).reshape(_BL)
    x = prop_atom_chirality.astype(jnp.int32).reshape(_BL)
    out = _sc_kernel(r, c, h, x,
                     W_in_ring.reshape(-1), W_charge.reshape(-1),
                     W_hybridization.reshape(-1), W_chirality.reshape(-1))
    return out.reshape(_B, _L, _D)


# speedup vs baseline: 36.0097x; 1.0296x over previous
import functools

import jax
import jax.numpy as jnp
from jax import lax
from jax.experimental import pallas as pl
from jax.experimental.pallas import tpu as pltpu
from jax.experimental.pallas import tpu_sc as plsc

_B, _L, _D = 1024, 200, 128
_BL = _B * _L
_N_RING, _N_CHARGE, _N_HYB, _N_CHIR = 3, 4, 9, 5
_NCOMB_PAD = 544

_NC, _NS = 2, 16
_NW = _NC * _NS
_PER_W = _BL // _NW
_CHUNK = 128
_NCHUNKS = _PER_W // _CHUNK
_ROWS_PER_TILE = _NCOMB_PAD // _NS
_NBUF = 2


def _sc_body(ring_hbm, charge_hbm, hyb_hbm, chir_hbm,
             wr_hbm, wc_hbm, wh_hbm, wx_hbm,
             out_hbm,
             idx4_v, cidx_v, wr_v, wc_v, wh_v, wx_v, tmp_v, rows_v, wcomb_sh,
             isem, g0, g1, s0, s1):
    cid = lax.axis_index("c")
    sid = lax.axis_index("s")
    wid = sid * _NC + cid
    tile_base = wid * _PER_W

    idx_cp = [
        pltpu.make_async_copy(src.at[pl.ds(tile_base, _PER_W)],
                              idx4_v.at[i], isem)
        for i, src in enumerate((ring_hbm, charge_hbm, hyb_hbm, chir_hbm))
    ]
    for cp in idx_cp:
        cp.start()

    tabs = [wr_v, wc_v, wh_v, wx_v]
    for dst, s_ in zip(tabs, (wr_hbm, wc_hbm, wh_hbm, wx_hbm)):
        pltpu.sync_copy(s_, dst.at[pl.ds(0, s_.shape[0])])

    def build_row(jl, carry):
        j = sid * _ROWS_PER_TILE + jl
        r = j // (_N_CHARGE * _N_HYB * _N_CHIR)
        c = (j // (_N_HYB * _N_CHIR)) % _N_CHARGE
        h = (j // _N_CHIR) % _N_HYB
        x = j % _N_CHIR
        for k in range(_D // 16):
            v = (wr_v[pl.ds(r * _D + 16 * k, 16)]
                 + wc_v[pl.ds(c * _D + 16 * k, 16)]
                 + wh_v[pl.ds(h * _D + 16 * k, 16)]
                 + wx_v[pl.ds(x * _D + 16 * k, 16)])
            tmp_v[jl, pl.ds(16 * k, 16)] = v
        return carry

    lax.fori_loop(0, _ROWS_PER_TILE, build_row, 0)
    pltpu.sync_copy(
        tmp_v, wcomb_sh.at[pl.ds(sid * _ROWS_PER_TILE, _ROWS_PER_TILE)])

    for cp in idx_cp:
        cp.wait()

    def fuse_chunk(g):
        for k in range(_CHUNK // 16):
            s = pl.ds(g * _CHUNK + k * 16, 16)
            cidx_v[s] = ((idx4_v[0, s] * (_N_CHARGE * _N_HYB * _N_CHIR))
                         + (idx4_v[1, s] * (_N_HYB * _N_CHIR))
                         + (idx4_v[2, s] * _N_CHIR)
                         + idx4_v[3, s])

    fuse_chunk(0)
    fuse_chunk(1)

    plsc.subcore_barrier()

    ssems = [s0, s1]
    gsems = [g0, g1]

    def scat(b, g):
        base = tile_base + g * _CHUNK
        return pltpu.make_async_copy(
            rows_v.at[b], out_hbm.at[pl.ds(base, _CHUNK)], ssems[b])

    def gath(b, g):
        return pltpu.make_async_copy(
            wcomb_sh.at[cidx_v.at[pl.ds(g * _CHUNK, _CHUNK)]],
            rows_v.at[b], gsems[b])

    def step(i, carry):
        for b in range(_NBUF):
            g = i * _NBUF + b

            @pl.when(i > 0)
            def _():
                scat(b, g - 2).wait()
                if b == 0:
                    gath(b, g).start()

            if b == 0:
                @pl.when(i == 0)
                def _():
                    gath(b, g).start()
            else:
                gath(b, g).start()

            @pl.when(g + 2 < _NCHUNKS)
            def _():
                fuse_chunk(g + 2)

            if b == 0:
                @pl.when(i > 0)
                def _():
                    gath(1, g - 1).wait()
                    scat(1, g - 1).start()
            else:
                gath(0, g - 1).wait()
                scat(0, g - 1).start()
        return carry

    lax.fori_loop(0, _NCHUNKS // _NBUF, step, 0)
    gath(1, _NCHUNKS - 1).wait()
    scat(1, _NCHUNKS - 1).start()
    scat(0, 0).wait()
    scat(1, 0).wait()


_sc_kernel = functools.partial(
    pl.kernel,
    out_type=jax.ShapeDtypeStruct((_BL, _D), jnp.float32),
    mesh=plsc.VectorSubcoreMesh(core_axis_name="c", subcore_axis_name="s"),
    scratch_types=[
        pltpu.VMEM((4, _PER_W), jnp.int32),
        pltpu.VMEM((_PER_W,), jnp.int32),
        pltpu.VMEM((16 * _D,), jnp.float32),
        pltpu.VMEM((16 * _D,), jnp.float32),
        pltpu.VMEM((16 * _D,), jnp.float32),
        pltpu.VMEM((16 * _D,), jnp.float32),
        pltpu.VMEM((_ROWS_PER_TILE, _D), jnp.float32),
        pltpu.VMEM((_NBUF, _CHUNK, _D), jnp.float32),
        pltpu.VMEM_SHARED((_NCOMB_PAD, _D), jnp.float32),
        pltpu.SemaphoreType.DMA,
        pltpu.SemaphoreType.DMA,
        pltpu.SemaphoreType.DMA,
        pltpu.SemaphoreType.DMA,
        pltpu.SemaphoreType.DMA,
    ],
)(_sc_body)


@jax.jit
def kernel(prop_atom_in_ring, prop_atom_charge, prop_atom_hybridization,
           prop_atom_chirality, W_in_ring, W_charge, W_hybridization,
           W_chirality):
    r = prop_atom_in_ring.astype(jnp.int32).reshape(_BL)
    c = prop_atom_charge.astype(jnp.int32).reshape(_BL)
    h = prop_atom_hybridization.astype(jnp.int32).reshape(_BL)
    x = prop_atom_chirality.astype(jnp.int32).reshape(_BL)
    out = _sc_kernel(r, c, h, x,
                     W_in_ring.reshape(-1), W_charge.reshape(-1),
                     W_hybridization.reshape(-1), W_chirality.reshape(-1))
    return out.reshape(_B, _L, _D)
